# baseline (speedup 1.0000x reference)
import jax
import jax.numpy as jnp
from jax.experimental import pallas as pl
from jax.experimental.pallas import tpu as pltpu

_BLOCK_COLS = 28672


def _tc_body(id_ref, tt_ref, x_ref, o_ref):
    ct = id_ref[0]
    tt = tt_ref[...]
    lane = jax.lax.broadcasted_iota(jnp.int32, tt.shape, 1)
    col = jnp.sum(jnp.where(lane == ct, tt, 0.0), axis=1, keepdims=True)
    o_ref[...] = x_ref[...] + col


def kernel(x, cell_type_id, table):
    n, d = x.shape
    xt = x.T
    tt = table.T
    ct = cell_type_id.astype(jnp.int32)
    grid = pl.cdiv(n, _BLOCK_COLS)

    outt = pl.pallas_call(
        _tc_body,
        grid=(grid,),
        in_specs=[
            pl.BlockSpec(memory_space=pltpu.SMEM),
            pl.BlockSpec((d, tt.shape[1]), lambda i: (0, 0)),
            pl.BlockSpec((d, _BLOCK_COLS), lambda i: (0, i)),
        ],
        out_specs=pl.BlockSpec((d, _BLOCK_COLS), lambda i: (0, i)),
        out_shape=jax.ShapeDtypeStruct((d, n), jnp.float32),
        compiler_params=pltpu.CompilerParams(
            dimension_semantics=("parallel",),
        ),
    )(ct, tt, xt)
    return outt.T


# speedup vs baseline: 1.0359x; 1.0359x over previous
import jax
import jax.numpy as jnp
from jax.experimental import pallas as pl
from jax.experimental.pallas import tpu as pltpu

_BLOCK_COLS = 49152


def _tc_body(id_ref, tt_ref, x_ref, o_ref):
    ct = id_ref[0]
    tt = tt_ref[...]
    lane = jax.lax.broadcasted_iota(jnp.int32, tt.shape, 1)
    col = jnp.sum(jnp.where(lane == ct, tt, 0.0), axis=1, keepdims=True)
    o_ref[...] = x_ref[...] + col


def kernel(x, cell_type_id, table):
    n, d = x.shape
    xt = x.T
    tt = table.T
    ct = cell_type_id.astype(jnp.int32)
    grid = pl.cdiv(n, _BLOCK_COLS)

    outt = pl.pallas_call(
        _tc_body,
        grid=(grid,),
        in_specs=[
            pl.BlockSpec(memory_space=pltpu.SMEM),
            pl.BlockSpec((d, tt.shape[1]), lambda i: (0, 0)),
            pl.BlockSpec((d, _BLOCK_COLS), lambda i: (0, i)),
        ],
        out_specs=pl.BlockSpec((d, _BLOCK_COLS), lambda i: (0, i)),
        out_shape=jax.ShapeDtypeStruct((d, n), jnp.float32),
        compiler_params=pltpu.CompilerParams(
            dimension_semantics=("parallel",),
        ),
    )(ct, tt, xt)
    return outt.T
